# baseline (speedup 1.0000x reference)
import jax
import jax.numpy as jnp
from jax.experimental import pallas as pl

N_NODES = 10000
WINDOW = 4
NTOT = N_NODES * WINDOW
IN_CH = 128
HID = 128


def _gcn_conv(x, W, b, src, dst, ew, n):
    h = x @ W
    loop = jnp.arange(n, dtype=src.dtype)
    src2 = jnp.concatenate([src, loop])
    dst2 = jnp.concatenate([dst, loop])
    ew2 = jnp.concatenate([ew, jnp.ones((n,), x.dtype)])
    deg = jnp.zeros((n,), x.dtype).at[dst2].add(ew2)
    dinv = jnp.where(deg > 0, jax.lax.rsqrt(jnp.maximum(deg, 1e-12)), 0.0)
    norm = dinv[src2] * ew2 * dinv[dst2]
    out = jnp.zeros((n, W.shape[1]), x.dtype).at[dst2].add(h[src2] * norm[:, None])
    return out + b


def _batchnorm(x, gamma, beta, eps=1e-5):
    mu = x.mean(0)
    var = x.var(0)
    return (x - mu) / jnp.sqrt(var + eps) * gamma + beta


def _lstm(seq, Wih, Whh, bih, bhh, hsize):
    B = seq.shape[1]
    h0 = jnp.zeros((B, hsize), seq.dtype)
    c0 = jnp.zeros((B, hsize), seq.dtype)
    def step(carry, xt):
        h, c = carry
        gates = xt @ Wih.T + h @ Whh.T + bih + bhh
        i, f, g, o = jnp.split(gates, 4, axis=-1)
        i = jax.nn.sigmoid(i); f = jax.nn.sigmoid(f)
        g = jnp.tanh(g); o = jax.nn.sigmoid(o)
        c = f * c + i * g
        h = o * jnp.tanh(c)
        return (h, c), h
    (hn, cn), ys = jax.lax.scan(step, (h0, c0), seq)
    return ys, hn


def kernel(x, edge_index, edge_weight, W1, b1, W2, b2, gamma, beta, Wih1, Whh1, bih1, bhh1, Wih2, Whh2, bih2, bhh2):
    src = edge_index[0]
    dst = edge_index[1]
    skip = x.reshape(-1, WINDOW, N_NODES, IN_CH)
    skip = jnp.transpose(skip, (0, 2, 1, 3)).reshape(-1, WINDOW, IN_CH)
    overlap = [skip[:, 0, :]]
    for l in range(1, WINDOW):
        overlap.append(skip[:, l, IN_CH - 1][:, None])
    skip = jnp.concatenate(overlap, axis=1)
    h1 = _batchnorm(jax.nn.relu(_gcn_conv(x, W1, b1, src, dst, edge_weight, NTOT)), gamma, beta)
    h2 = _batchnorm(jax.nn.relu(_gcn_conv(h1, W2, b2, src, dst, edge_weight, NTOT)), gamma, beta)
    xc = jnp.concatenate([h1, h2], axis=1)
    xc = xc.reshape(-1, WINDOW, N_NODES, 2 * HID)
    xc = jnp.transpose(xc, (1, 0, 2, 3)).reshape(WINDOW, -1, 2 * HID)
    ys1, hn1 = _lstm(xc, Wih1, Whh1, bih1, bhh1, HID)
    ys2, hn2 = _lstm(ys1, Wih2, Whh2, bih2, bhh2, HID)
    out = jnp.concatenate([hn1, hn2, skip], axis=1)
    return out


# speedup vs baseline: 2.7942x; 2.7942x over previous
import functools

import jax
import jax.numpy as jnp
from jax.experimental import pallas as pl
from jax.experimental.pallas import tpu as pltpu

N_NODES = 10000
WINDOW = 4
NTOT = N_NODES * WINDOW
IN_CH = 128
HID = 128
E = 600000

ROW_BLK = 2000
LSTM_BLK = 2000


def _dinv_blk(d0_ref, d1_ref):
    return jax.lax.rsqrt(1.0 + d0_ref[0, 0, :] + d1_ref[0, 0, :])


_DEG_SPEC = lambda: pl.BlockSpec((1, 1, ROW_BLK), lambda i: (i, 0, 0))


def _mm_scale_body(x_ref, w_ref, d0_ref, d1_ref, hh_ref):
    dinv = _dinv_blk(d0_ref, d1_ref)
    h = jnp.dot(x_ref[...], w_ref[...], preferred_element_type=jnp.float32)
    hh_ref[...] = h * dinv[:, None]


def _mm_scale(x, w, d0, d1):
    n = x.shape[0]
    grid = n // ROW_BLK
    return pl.pallas_call(
        _mm_scale_body,
        grid=(grid,),
        in_specs=[
            pl.BlockSpec((ROW_BLK, x.shape[1]), lambda i: (i, 0)),
            pl.BlockSpec((x.shape[1], w.shape[1]), lambda i: (0, 0)),
            _DEG_SPEC(),
            _DEG_SPEC(),
        ],
        out_specs=pl.BlockSpec((ROW_BLK, w.shape[1]), lambda i: (i, 0)),
        out_shape=jax.ShapeDtypeStruct((n, w.shape[1]), jnp.float32),
    )(x, w, d0, d1)


def _relu_stats_body(acc_ref, hh_ref, d0_ref, d1_ref, b_ref, r_ref, stats_ref):
    dinv = _dinv_blk(d0_ref, d1_ref)
    y = dinv[:, None] * (acc_ref[...] + hh_ref[...]) + b_ref[0, :][None, :]
    r = jnp.maximum(y, 0.0)
    r_ref[...] = r

    @pl.when(pl.program_id(0) == 0)
    def _():
        stats_ref[...] = jnp.zeros_like(stats_ref)

    s = jnp.sum(r, axis=0)
    sq = jnp.sum(r * r, axis=0)
    stats_ref[...] += jnp.stack([s, sq], axis=0)


def _relu_stats(acc, hh, d0, d1, b):
    n, d = acc.shape
    grid = n // ROW_BLK
    return pl.pallas_call(
        _relu_stats_body,
        grid=(grid,),
        in_specs=[
            pl.BlockSpec((ROW_BLK, d), lambda i: (i, 0)),
            pl.BlockSpec((ROW_BLK, d), lambda i: (i, 0)),
            _DEG_SPEC(),
            _DEG_SPEC(),
            pl.BlockSpec((1, d), lambda i: (0, 0)),
        ],
        out_specs=[
            pl.BlockSpec((ROW_BLK, d), lambda i: (i, 0)),
            pl.BlockSpec((2, d), lambda i: (0, 0)),
        ],
        out_shape=[
            jax.ShapeDtypeStruct((n, d), jnp.float32),
            jax.ShapeDtypeStruct((2, d), jnp.float32),
        ],
    )(acc, hh, d0, d1, b.reshape(1, d))


def _bn_mm_body(r_ref, sc_ref, sh_ref, w_ref, d0_ref, d1_ref, bn_ref, hh_ref):
    bn = r_ref[...] * sc_ref[0, :][None, :] + sh_ref[0, :][None, :]
    bn_ref[...] = bn
    dinv = _dinv_blk(d0_ref, d1_ref)
    h = jnp.dot(bn, w_ref[...], preferred_element_type=jnp.float32)
    hh_ref[...] = h * dinv[:, None]


def _bn_mm(r, scale, shift, w, d0, d1):
    n, d = r.shape
    grid = n // ROW_BLK
    return pl.pallas_call(
        _bn_mm_body,
        grid=(grid,),
        in_specs=[
            pl.BlockSpec((ROW_BLK, d), lambda i: (i, 0)),
            pl.BlockSpec((1, d), lambda i: (0, 0)),
            pl.BlockSpec((1, d), lambda i: (0, 0)),
            pl.BlockSpec((d, w.shape[1]), lambda i: (0, 0)),
            _DEG_SPEC(),
            _DEG_SPEC(),
        ],
        out_specs=[
            pl.BlockSpec((ROW_BLK, d), lambda i: (i, 0)),
            pl.BlockSpec((ROW_BLK, w.shape[1]), lambda i: (i, 0)),
        ],
        out_shape=[
            jax.ShapeDtypeStruct((n, d), jnp.float32),
            jax.ShapeDtypeStruct((n, w.shape[1]), jnp.float32),
        ],
    )(r, scale.reshape(1, d), shift.reshape(1, d), w, d0, d1)


def _lstm_body(bn1_ref, r2_ref, sc2_ref, sh2_ref, wia_ref, wib_ref, wh1_ref,
               wi2_ref, wh2_ref, b1_ref, b2_ref, hn1_ref, hn2_ref):
    nb = bn1_ref.shape[1]
    h1 = jnp.zeros((nb, HID), jnp.float32)
    c1 = jnp.zeros((nb, HID), jnp.float32)
    h2 = jnp.zeros((nb, HID), jnp.float32)
    c2 = jnp.zeros((nb, HID), jnp.float32)
    sc2 = sc2_ref[0, :][None, :]
    sh2 = sh2_ref[0, :][None, :]
    for t in range(WINDOW):
        xt1 = bn1_ref[t]
        xt2 = r2_ref[t] * sc2 + sh2
        g1 = (jnp.dot(xt1, wia_ref[...], preferred_element_type=jnp.float32)
              + jnp.dot(xt2, wib_ref[...], preferred_element_type=jnp.float32)
              + jnp.dot(h1, wh1_ref[...], preferred_element_type=jnp.float32)
              + b1_ref[0, :][None, :])
        i1 = jax.nn.sigmoid(g1[:, 0 * HID:1 * HID])
        f1 = jax.nn.sigmoid(g1[:, 1 * HID:2 * HID])
        gg1 = jnp.tanh(g1[:, 2 * HID:3 * HID])
        o1 = jax.nn.sigmoid(g1[:, 3 * HID:4 * HID])
        c1 = f1 * c1 + i1 * gg1
        h1 = o1 * jnp.tanh(c1)
        g2 = (jnp.dot(h1, wi2_ref[...], preferred_element_type=jnp.float32)
              + jnp.dot(h2, wh2_ref[...], preferred_element_type=jnp.float32)
              + b2_ref[0, :][None, :])
        i2 = jax.nn.sigmoid(g2[:, 0 * HID:1 * HID])
        f2 = jax.nn.sigmoid(g2[:, 1 * HID:2 * HID])
        gg2 = jnp.tanh(g2[:, 2 * HID:3 * HID])
        o2 = jax.nn.sigmoid(g2[:, 3 * HID:4 * HID])
        c2 = f2 * c2 + i2 * gg2
        h2 = o2 * jnp.tanh(c2)
    hn1_ref[...] = h1
    hn2_ref[...] = h2


def _lstm_fused(bn1, r2, scale2, shift2, Wih1, Whh1, bsum1, Wih2, Whh2, bsum2):
    bn1v = bn1.reshape(WINDOW, N_NODES, HID)
    r2v = r2.reshape(WINDOW, N_NODES, HID)
    wia = Wih1.T[:HID]
    wib = Wih1.T[HID:]
    wh1 = Whh1.T
    wi2 = Wih2.T
    wh2 = Whh2.T
    grid = N_NODES // LSTM_BLK
    return pl.pallas_call(
        _lstm_body,
        grid=(grid,),
        in_specs=[
            pl.BlockSpec((WINDOW, LSTM_BLK, HID), lambda i: (0, i, 0)),
            pl.BlockSpec((WINDOW, LSTM_BLK, HID), lambda i: (0, i, 0)),
            pl.BlockSpec((1, HID), lambda i: (0, 0)),
            pl.BlockSpec((1, HID), lambda i: (0, 0)),
            pl.BlockSpec((HID, 4 * HID), lambda i: (0, 0)),
            pl.BlockSpec((HID, 4 * HID), lambda i: (0, 0)),
            pl.BlockSpec((HID, 4 * HID), lambda i: (0, 0)),
            pl.BlockSpec((HID, 4 * HID), lambda i: (0, 0)),
            pl.BlockSpec((HID, 4 * HID), lambda i: (0, 0)),
            pl.BlockSpec((1, 4 * HID), lambda i: (0, 0)),
            pl.BlockSpec((1, 4 * HID), lambda i: (0, 0)),
        ],
        out_specs=[
            pl.BlockSpec((LSTM_BLK, HID), lambda i: (i, 0)),
            pl.BlockSpec((LSTM_BLK, HID), lambda i: (i, 0)),
        ],
        out_shape=[
            jax.ShapeDtypeStruct((N_NODES, HID), jnp.float32),
            jax.ShapeDtypeStruct((N_NODES, HID), jnp.float32),
        ],
    )(bn1v, r2v, scale2.reshape(1, HID), shift2.reshape(1, HID),
      wia, wib, wh1, wi2, wh2, bsum1.reshape(1, 4 * HID), bsum2.reshape(1, 4 * HID))


def _deg_partials(dst, ew):
    deg = jnp.zeros((NTOT,), jnp.float32).at[dst].add(ew)
    return deg, jnp.zeros_like(deg)


def _aggregate(hh, src, dst, ew):
    g = hh[src] * ew[:, None]
    return jnp.zeros((NTOT, HID), jnp.float32).at[dst].add(g)


def _bn_coeffs(stats, gamma, beta, n, eps=1e-5):
    mu = stats[0] / n
    var = stats[1] / n - mu * mu
    scale = gamma / jnp.sqrt(var + eps)
    shift = beta - mu * scale
    return scale, shift


def kernel(x, edge_index, edge_weight, W1, b1, W2, b2, gamma, beta,
           Wih1, Whh1, bih1, bhh1, Wih2, Whh2, bih2, bhh2):
    src = edge_index[0]
    dst = edge_index[1]

    dp0, dp1 = _deg_partials(dst, edge_weight)
    d0 = dp0.reshape(NTOT // ROW_BLK, 1, ROW_BLK)
    d1 = dp1.reshape(NTOT // ROW_BLK, 1, ROW_BLK)

    hh1 = _mm_scale(x, W1, d0, d1)
    acc1 = _aggregate(hh1, src, dst, edge_weight)
    r1, stats1 = _relu_stats(acc1, hh1, d0, d1, b1)
    scale1, shift1 = _bn_coeffs(stats1, gamma, beta, NTOT)

    bn1, hh2 = _bn_mm(r1, scale1, shift1, W2, d0, d1)
    acc2 = _aggregate(hh2, src, dst, edge_weight)
    r2, stats2 = _relu_stats(acc2, hh2, d0, d1, b2)
    scale2, shift2 = _bn_coeffs(stats2, gamma, beta, NTOT)

    hn1, hn2 = _lstm_fused(bn1, r2, scale2, shift2,
                           Wih1, Whh1, bih1 + bhh1, Wih2, Whh2, bih2 + bhh2)

    skip = jnp.concatenate(
        [x[:N_NODES]]
        + [x[l * N_NODES:(l + 1) * N_NODES, IN_CH - 1:IN_CH] for l in range(1, WINDOW)],
        axis=1)

    return jnp.concatenate([hn1, hn2, skip], axis=1)
